# Initial kernel scaffold; baseline (speedup 1.0000x reference)
import jax
import jax.numpy as jnp
from jax.experimental import pallas as pl


def kernel(x, qkv_wt, qkv_b, o_wt, o_b):
    raise NotImplementedError("write your pallas kernel here")


# speedup vs baseline: 2.1132x; 2.1132x over previous
import functools

import jax
import jax.numpy as jnp
from jax import lax
from jax.experimental import pallas as pl
from jax.experimental.pallas import tpu as pltpu


def _mha_kernel(x_ref, wq_ref, wk_ref, wv_ref, bq_ref, bk_ref, bv_ref,
                wo_ref, ob_ref, out_ref, attn_ref, *, g, dk):
    j = pl.program_id(1)

    x = x_ref[0].astype(jnp.bfloat16)
    q = (jnp.dot(x, wq_ref[...], preferred_element_type=jnp.float32)
         + bq_ref[...]).astype(jnp.bfloat16)
    k = (jnp.dot(x, wk_ref[...], preferred_element_type=jnp.float32)
         + bk_ref[...]).astype(jnp.bfloat16)
    v = (jnp.dot(x, wv_ref[...], preferred_element_type=jnp.float32)
         + bv_ref[...]).astype(jnp.bfloat16)

    ctx_parts = []
    for h in range(g):
        sl = slice(h * dk, (h + 1) * dk)
        s = lax.dot_general(q[:, sl], k[:, sl], (((1,), (1,)), ((), ())),
                            preferred_element_type=jnp.float32)
        row_max = jnp.max(s, axis=-1, keepdims=True)
        e = jnp.exp(s - row_max)
        denom = jnp.sum(e, axis=-1, keepdims=True)
        p = e * (1.0 / denom)
        attn_ref[0, h] = p
        ctx_parts.append(jnp.dot(p.astype(jnp.bfloat16), v[:, sl],
                                 preferred_element_type=jnp.float32))

    ctx = jnp.concatenate(ctx_parts, axis=1).astype(jnp.bfloat16)
    partial = jnp.dot(ctx, wo_ref[...], preferred_element_type=jnp.float32)

    @pl.when(j == 0)
    def _init():
        out_ref[0] = partial + ob_ref[...]

    @pl.when(j != 0)
    def _acc():
        out_ref[0] += partial


def kernel(x, qkv_wt, qkv_b, o_wt, o_b):
    bs, L, D = x.shape
    dk = 64
    nh = D // dk
    g = 8
    nj = nh // g
    gd = g * dk

    b2 = qkv_b.reshape(1, 3 * D).astype(jnp.float32)
    ob2 = o_b.reshape(1, D).astype(jnp.float32)

    out, attn = pl.pallas_call(
        functools.partial(_mha_kernel, g=g, dk=dk),
        out_shape=(
            jax.ShapeDtypeStruct((bs, L, D), jnp.float32),
            jax.ShapeDtypeStruct((bs, nh, L, L), jnp.float32),
        ),
        grid=(bs, nj),
        in_specs=[
            pl.BlockSpec((1, L, D), lambda b, j: (b, 0, 0)),
            pl.BlockSpec((D, gd), lambda b, j: (0, j)),
            pl.BlockSpec((D, gd), lambda b, j: (0, j + nj)),
            pl.BlockSpec((D, gd), lambda b, j: (0, j + 2 * nj)),
            pl.BlockSpec((1, gd), lambda b, j: (0, j)),
            pl.BlockSpec((1, gd), lambda b, j: (0, j + nj)),
            pl.BlockSpec((1, gd), lambda b, j: (0, j + 2 * nj)),
            pl.BlockSpec((gd, D), lambda b, j: (j, 0)),
            pl.BlockSpec((1, D), lambda b, j: (0, 0)),
        ],
        out_specs=(
            pl.BlockSpec((1, L, D), lambda b, j: (b, 0, 0)),
            pl.BlockSpec((1, g, L, L), lambda b, j: (b, j, 0, 0)),
        ),
        compiler_params=pltpu.CompilerParams(
            dimension_semantics=("parallel", "arbitrary"),
            vmem_limit_bytes=56 * 1024 * 1024,
        ),
    )(x, qkv_wt, qkv_wt, qkv_wt, b2, b2, b2, o_wt, ob2)
    return out, attn
